# Initial kernel scaffold; baseline (speedup 1.0000x reference)
import jax
import jax.numpy as jnp
from jax.experimental import pallas as pl


def kernel(a, b, c):
    raise NotImplementedError("write your pallas kernel here")


# speedup vs baseline: 3.2993x; 3.2993x over previous
import functools

import jax
import jax.numpy as jnp
from jax import lax
from jax.experimental import pallas as pl
from jax.experimental.pallas import tpu as pltpu
from jax.experimental.pallas import tpu_sc as plsc

_L = 16
_NC = 2
_NS = 16
_NW = _NC * _NS

_N_TOTAL = 64 * 32768
_EW = _N_TOTAL // _NW
_CHUNK = 16384
_NCHUNK = _EW // _CHUNK

_mesh = plsc.VectorSubcoreMesh(core_axis_name="c", subcore_axis_name="s")


@functools.partial(
    pl.kernel,
    mesh=_mesh,
    out_type=jax.ShapeDtypeStruct((_N_TOTAL,), jnp.float32),
    scratch_types=[
        pltpu.VMEM((_CHUNK,), jnp.float32),
        pltpu.VMEM((_CHUNK,), jnp.float32),
        pltpu.VMEM((_CHUNK,), jnp.float32),
        pltpu.VMEM((_CHUNK,), jnp.float32),
    ],
)
def _median_sc(a_hbm, b_hbm, c_hbm, out_hbm, av, bv, cv, ov):
    wid = lax.axis_index("s") * _NC + lax.axis_index("c")
    base = wid * _EW

    def chunk_body(i, carry):
        off = base + i * _CHUNK
        pltpu.sync_copy(a_hbm.at[pl.ds(off, _CHUNK)], av)
        pltpu.sync_copy(b_hbm.at[pl.ds(off, _CHUNK)], bv)
        pltpu.sync_copy(c_hbm.at[pl.ds(off, _CHUNK)], cv)

        def vec_body(j, c2):
            s = j * _L
            x = av[pl.ds(s, _L)]
            y = bv[pl.ds(s, _L)]
            z = cv[pl.ds(s, _L)]
            lo = jnp.minimum(x, y)
            hi = jnp.maximum(x, y)
            ov[pl.ds(s, _L)] = jnp.maximum(lo, jnp.minimum(hi, z))
            return c2

        lax.fori_loop(0, _CHUNK // _L, vec_body, 0, unroll=8)
        pltpu.sync_copy(ov, out_hbm.at[pl.ds(off, _CHUNK)])
        return carry

    lax.fori_loop(0, _NCHUNK, chunk_body, 0)


def kernel(a, b, c):
    out = _median_sc(a.reshape(-1), b.reshape(-1), c.reshape(-1))
    return out.reshape(a.shape)


# speedup vs baseline: 5.4242x; 1.6440x over previous
import functools

import jax
import jax.numpy as jnp
from jax import lax
from jax.experimental import pallas as pl
from jax.experimental.pallas import tpu as pltpu
from jax.experimental.pallas import tpu_sc as plsc

_L = 16
_NC = 2
_NS = 16
_NW = _NC * _NS

_ROWS = 64
_COLS = 32768
_RB = 8
_NRB = _ROWS // _RB
_WPR = _NW // _NRB
_WCOLS = _COLS // _WPR
_CHUNK = 2048
_NCHUNK = _WCOLS // _CHUNK

_mesh = plsc.VectorSubcoreMesh(core_axis_name="c", subcore_axis_name="s")


@functools.partial(
    pl.kernel,
    mesh=_mesh,
    out_type=jax.ShapeDtypeStruct((_ROWS, _COLS), jnp.float32),
    scratch_types=[
        pltpu.VMEM((_RB, _CHUNK), jnp.float32),
        pltpu.VMEM((_RB, _CHUNK), jnp.float32),
        pltpu.VMEM((_RB, _CHUNK), jnp.float32),
        pltpu.VMEM((_RB, _CHUNK), jnp.float32),
    ],
)
def _median_sc(a_hbm, b_hbm, c_hbm, out_hbm, av, bv, cv, ov):
    wid = lax.axis_index("s") * _NC + lax.axis_index("c")
    r0 = (wid // _WPR) * _RB
    c0 = (wid % _WPR) * _WCOLS

    def chunk_body(i, carry):
        off = c0 + i * _CHUNK
        pltpu.sync_copy(a_hbm.at[pl.ds(r0, _RB), pl.ds(off, _CHUNK)], av)
        pltpu.sync_copy(b_hbm.at[pl.ds(r0, _RB), pl.ds(off, _CHUNK)], bv)
        pltpu.sync_copy(c_hbm.at[pl.ds(r0, _RB), pl.ds(off, _CHUNK)], cv)

        for r in range(_RB):
            def vec_body(j, c2, r=r):
                s = j * _L
                x = av[r, pl.ds(s, _L)]
                y = bv[r, pl.ds(s, _L)]
                z = cv[r, pl.ds(s, _L)]
                lo = jnp.minimum(x, y)
                hi = jnp.maximum(x, y)
                ov[r, pl.ds(s, _L)] = jnp.maximum(lo, jnp.minimum(hi, z))
                return c2

            lax.fori_loop(0, _CHUNK // _L, vec_body, 0, unroll=8)

        pltpu.sync_copy(ov, out_hbm.at[pl.ds(r0, _RB), pl.ds(off, _CHUNK)])
        return carry

    lax.fori_loop(0, _NCHUNK, chunk_body, 0)


def kernel(a, b, c):
    return _median_sc(a, b, c)


# speedup vs baseline: 5.8645x; 1.0812x over previous
import functools

import jax
import jax.numpy as jnp
from jax import lax
from jax.experimental import pallas as pl
from jax.experimental.pallas import tpu as pltpu
from jax.experimental.pallas import tpu_sc as plsc

_L = 16
_NC = 2
_NS = 16
_NW = _NC * _NS

_ROWS = 64
_COLS = 32768
_RB = 8
_NRB = _ROWS // _RB
_WPR = _NW // _NRB
_WCOLS = _COLS // _WPR
_CHUNK = 1024
_NCHUNK = _WCOLS // _CHUNK

_mesh = plsc.VectorSubcoreMesh(core_axis_name="c", subcore_axis_name="s")


@functools.partial(
    pl.kernel,
    mesh=_mesh,
    out_type=jax.ShapeDtypeStruct((_ROWS, _COLS), jnp.float32),
    scratch_types=(
        [pltpu.VMEM((_RB, _CHUNK), jnp.float32)] * 8
        + [pltpu.SemaphoreType.DMA] * 4
    ),
)
def _median_sc(a_hbm, b_hbm, c_hbm, out_hbm,
               av0, av1, bv0, bv1, cv0, cv1, ov0, ov1,
               si0, si1, so0, so1):
    wid = lax.axis_index("s") * _NC + lax.axis_index("c")
    r0 = (wid // _WPR) * _RB
    c0 = (wid % _WPR) * _WCOLS

    ins = [(av0, bv0, cv0, si0), (av1, bv1, cv1, si1)]
    outs = [(ov0, so0), (ov1, so1)]

    def hbm_slice(ref, i):
        return ref.at[pl.ds(r0, _RB), pl.ds(c0 + i * _CHUNK, _CHUNK)]

    def start_in(i):
        av, bv, cv, si = ins[i % 2]
        return [
            pltpu.async_copy(hbm_slice(a_hbm, i), av, si),
            pltpu.async_copy(hbm_slice(b_hbm, i), bv, si),
            pltpu.async_copy(hbm_slice(c_hbm, i), cv, si),
        ]

    h_in = [None] * _NCHUNK
    h_out = [None] * _NCHUNK
    h_in[0] = start_in(0)

    for i in range(_NCHUNK):
        if i + 1 < _NCHUNK:
            h_in[i + 1] = start_in(i + 1)
        for h in h_in[i]:
            h.wait()
        av, bv, cv, _ = ins[i % 2]
        ov, so = outs[i % 2]
        if i >= 2:
            h_out[i - 2].wait()

        for r in range(_RB):
            def vec_body(j, c2, r=r):
                s = j * _L
                x = av[r, pl.ds(s, _L)]
                y = bv[r, pl.ds(s, _L)]
                z = cv[r, pl.ds(s, _L)]
                lo = jnp.minimum(x, y)
                hi = jnp.maximum(x, y)
                ov[r, pl.ds(s, _L)] = jnp.maximum(lo, jnp.minimum(hi, z))
                return c2

            lax.fori_loop(0, _CHUNK // _L, vec_body, 0, unroll=8)

        h_out[i] = pltpu.async_copy(ov, hbm_slice(out_hbm, i), so)

    h_out[_NCHUNK - 2].wait()
    h_out[_NCHUNK - 1].wait()


def kernel(a, b, c):
    return _median_sc(a, b, c)


# speedup vs baseline: 9.4831x; 1.6170x over previous
import functools

import jax
import jax.numpy as jnp
from jax import lax
from jax.experimental import pallas as pl
from jax.experimental.pallas import tpu as pltpu
from jax.experimental.pallas import tpu_sc as plsc

_L = 16
_NC = 2
_NS = 16
_NW = _NC * _NS

_ROWS = 64
_COLS = 32768
_RB = 8
_NRB = _ROWS // _RB
_WPR = _NW // _NRB
_WCOLS = _COLS // _WPR
_CHUNK = 1024
_NCHUNK = _WCOLS // _CHUNK
_NPAIR = _NCHUNK // 2

_mesh = plsc.VectorSubcoreMesh(core_axis_name="c", subcore_axis_name="s")


@functools.partial(
    pl.kernel,
    mesh=_mesh,
    out_type=jax.ShapeDtypeStruct((_ROWS, _COLS), jnp.float32),
    scratch_types=(
        [pltpu.VMEM((_RB, _CHUNK), jnp.float32)] * 8
        + [pltpu.SemaphoreType.DMA] * 4
    ),
)
def _median_sc(a_hbm, b_hbm, c_hbm, out_hbm,
               av0, av1, bv0, bv1, cv0, cv1, ov0, ov1,
               si0, si1, so0, so1):
    wid = lax.axis_index("s") * _NC + lax.axis_index("c")
    r0 = (wid // _WPR) * _RB
    c0 = (wid % _WPR) * _WCOLS

    ins = [(av0, bv0, cv0, si0), (av1, bv1, cv1, si1)]
    outs = [(ov0, so0), (ov1, so1)]

    def hbm_slice(ref, ci):
        return ref.at[pl.ds(r0, _RB), pl.ds(c0 + ci * _CHUNK, _CHUNK)]

    def start_in(ci, p):
        av, bv, cv, si = ins[p]
        pltpu.async_copy(hbm_slice(a_hbm, ci), av, si)
        pltpu.async_copy(hbm_slice(b_hbm, ci), bv, si)
        pltpu.async_copy(hbm_slice(c_hbm, ci), cv, si)

    def wait_in(p):
        av, bv, cv, si = ins[p]
        pltpu.make_async_copy(hbm_slice(a_hbm, 0), av, si).wait()
        pltpu.make_async_copy(hbm_slice(b_hbm, 0), bv, si).wait()
        pltpu.make_async_copy(hbm_slice(c_hbm, 0), cv, si).wait()

    def wait_out(p):
        ov, so = outs[p]
        pltpu.make_async_copy(ov, hbm_slice(out_hbm, 0), so).wait()

    def compute(p, ci):
        av, bv, cv, _ = ins[p]
        ov, so = outs[p]
        for r in range(_RB):
            @plsc.parallel_loop(0, _CHUNK, _L, unroll=8)
            def vec_body(s, r=r):
                x = av[r, pl.ds(s, _L)]
                y = bv[r, pl.ds(s, _L)]
                z = cv[r, pl.ds(s, _L)]
                lo = jnp.minimum(x, y)
                hi = jnp.maximum(x, y)
                ov[r, pl.ds(s, _L)] = jnp.maximum(lo, jnp.minimum(hi, z))
        pltpu.async_copy(ov, hbm_slice(out_hbm, ci), so)

    start_in(0, 0)
    start_in(1, 1)

    def pair_body(k, carry):
        even = 2 * k
        wait_in(0)
        lax.cond(k > 0, lambda: wait_out(0), lambda: None)
        compute(0, even)
        lax.cond(k < _NPAIR - 1, lambda: start_in(even + 2, 0), lambda: None)
        wait_in(1)
        lax.cond(k > 0, lambda: wait_out(1), lambda: None)
        compute(1, even + 1)
        lax.cond(k < _NPAIR - 1, lambda: start_in(even + 3, 1), lambda: None)
        return carry

    lax.fori_loop(0, _NPAIR, pair_body, 0)
    wait_out(0)
    wait_out(1)


def kernel(a, b, c):
    return _median_sc(a, b, c)


# speedup vs baseline: 9.6864x; 1.0214x over previous
import functools

import jax
import jax.numpy as jnp
from jax import lax
from jax.experimental import pallas as pl
from jax.experimental.pallas import tpu as pltpu
from jax.experimental.pallas import tpu_sc as plsc

_L = 16
_NC = 2
_NS = 16
_NW = _NC * _NS

_ROWS = 64
_COLS = 32768
_RB = 8
_NRB = _ROWS // _RB
_WPR = _NW // _NRB
_WCOLS = _COLS // _WPR
_CHUNK = 1024
_NCHUNK = _WCOLS // _CHUNK
_NPAIR = _NCHUNK // 2

_mesh = plsc.VectorSubcoreMesh(core_axis_name="c", subcore_axis_name="s")


@functools.partial(
    pl.kernel,
    mesh=_mesh,
    out_type=jax.ShapeDtypeStruct((_ROWS, _COLS), jnp.float32),
    scratch_types=(
        [pltpu.VMEM((_RB, _CHUNK), jnp.float32)] * 8
        + [pltpu.SemaphoreType.DMA] * 4
    ),
)
def _median_sc(a_hbm, b_hbm, c_hbm, out_hbm,
               av0, av1, bv0, bv1, cv0, cv1, ov0, ov1,
               si0, si1, so0, so1):
    wid = lax.axis_index("s") * _NC + lax.axis_index("c")
    r0 = (wid // _WPR) * _RB
    c0 = (wid % _WPR) * _WCOLS

    ins = [(av0, bv0, cv0, si0), (av1, bv1, cv1, si1)]
    outs = [(ov0, so0), (ov1, so1)]

    def hbm_slice(ref, ci):
        return ref.at[pl.ds(r0, _RB), pl.ds(c0 + ci * _CHUNK, _CHUNK)]

    def start_in(ci, p):
        av, bv, cv, si = ins[p]
        pltpu.async_copy(hbm_slice(a_hbm, ci), av, si)
        pltpu.async_copy(hbm_slice(b_hbm, ci), bv, si)
        pltpu.async_copy(hbm_slice(c_hbm, ci), cv, si)

    def wait_in(p):
        av, bv, cv, si = ins[p]
        pltpu.make_async_copy(hbm_slice(a_hbm, 0), av, si).wait()
        pltpu.make_async_copy(hbm_slice(b_hbm, 0), bv, si).wait()
        pltpu.make_async_copy(hbm_slice(c_hbm, 0), cv, si).wait()

    def wait_out(p):
        ov, so = outs[p]
        pltpu.make_async_copy(ov, hbm_slice(out_hbm, 0), so).wait()

    def compute(p, ci):
        av, bv, cv, _ = ins[p]
        ov, so = outs[p]
        for r in range(_RB):
            @plsc.parallel_loop(0, _CHUNK, _L, unroll=4)
            def vec_body(s, r=r):
                x = av[r, pl.ds(s, _L)]
                y = bv[r, pl.ds(s, _L)]
                z = cv[r, pl.ds(s, _L)]
                lo = jnp.minimum(x, y)
                hi = jnp.maximum(x, y)
                ov[r, pl.ds(s, _L)] = jnp.maximum(lo, jnp.minimum(hi, z))
        pltpu.async_copy(ov, hbm_slice(out_hbm, ci), so)

    start_in(0, 0)
    start_in(1, 1)

    def pair_body(k, carry):
        even = 2 * k
        wait_in(0)
        lax.cond(k > 0, lambda: wait_out(0), lambda: None)
        compute(0, even)
        lax.cond(k < _NPAIR - 1, lambda: start_in(even + 2, 0), lambda: None)
        wait_in(1)
        lax.cond(k > 0, lambda: wait_out(1), lambda: None)
        compute(1, even + 1)
        lax.cond(k < _NPAIR - 1, lambda: start_in(even + 3, 1), lambda: None)
        return carry

    lax.fori_loop(0, _NPAIR, pair_body, 0)
    wait_out(0)
    wait_out(1)


def kernel(a, b, c):
    return _median_sc(a, b, c)


# speedup vs baseline: 9.8879x; 1.0208x over previous
import functools

import jax
import jax.numpy as jnp
from jax import lax
from jax.experimental import pallas as pl
from jax.experimental.pallas import tpu as pltpu
from jax.experimental.pallas import tpu_sc as plsc

_L = 16
_NC = 2
_NS = 16
_NW = _NC * _NS

_ROWS = 64
_COLS = 32768
_RB = 8
_NRB = _ROWS // _RB
_WPR = _NW // _NRB
_WCOLS = _COLS // _WPR
_CHUNK = 1024
_NCHUNK = _WCOLS // _CHUNK
_NPAIR = _NCHUNK // 2

_mesh = plsc.VectorSubcoreMesh(core_axis_name="c", subcore_axis_name="s")


@functools.partial(
    pl.kernel,
    mesh=_mesh,
    out_type=jax.ShapeDtypeStruct((_ROWS, _COLS), jnp.float32),
    scratch_types=(
        [pltpu.VMEM((_RB, _CHUNK), jnp.float32)] * 8
        + [pltpu.SemaphoreType.DMA] * 4
    ),
)
def _median_sc(a_hbm, b_hbm, c_hbm, out_hbm,
               av0, av1, bv0, bv1, cv0, cv1, ov0, ov1,
               si0, si1, so0, so1):
    wid = lax.axis_index("s") * _NC + lax.axis_index("c")
    r0 = (wid // _WPR) * _RB
    c0 = (wid % _WPR) * _WCOLS

    ins = [(av0, bv0, cv0, si0), (av1, bv1, cv1, si1)]
    outs = [(ov0, so0), (ov1, so1)]

    def hbm_slice(ref, ci):
        return ref.at[pl.ds(r0, _RB), pl.ds(c0 + ci * _CHUNK, _CHUNK)]

    def start_in(ci, p):
        av, bv, cv, si = ins[p]
        pltpu.async_copy(hbm_slice(a_hbm, ci), av, si)
        pltpu.async_copy(hbm_slice(b_hbm, ci), bv, si)
        pltpu.async_copy(hbm_slice(c_hbm, ci), cv, si)

    def wait_in(p):
        av, bv, cv, si = ins[p]
        pltpu.make_async_copy(hbm_slice(a_hbm, 0), av, si).wait()
        pltpu.make_async_copy(hbm_slice(b_hbm, 0), bv, si).wait()
        pltpu.make_async_copy(hbm_slice(c_hbm, 0), cv, si).wait()

    def wait_out(p):
        ov, so = outs[p]
        pltpu.make_async_copy(ov, hbm_slice(out_hbm, 0), so).wait()

    def compute(p, ci):
        av, bv, cv, _ = ins[p]
        ov, so = outs[p]
        @plsc.parallel_loop(0, _RB, 1)
        def row_body(r):
            @plsc.parallel_loop(0, _CHUNK, _L, unroll=4)
            def vec_body(s):
                x = av[r, pl.ds(s, _L)]
                y = bv[r, pl.ds(s, _L)]
                z = cv[r, pl.ds(s, _L)]
                lo = jnp.minimum(x, y)
                hi = jnp.maximum(x, y)
                ov[r, pl.ds(s, _L)] = jnp.maximum(lo, jnp.minimum(hi, z))
        pltpu.async_copy(ov, hbm_slice(out_hbm, ci), so)

    start_in(0, 0)
    start_in(1, 1)

    def pair_body(k, carry):
        even = 2 * k
        wait_in(0)
        lax.cond(k > 0, lambda: wait_out(0), lambda: None)
        compute(0, even)
        lax.cond(k < _NPAIR - 1, lambda: start_in(even + 2, 0), lambda: None)
        wait_in(1)
        lax.cond(k > 0, lambda: wait_out(1), lambda: None)
        compute(1, even + 1)
        lax.cond(k < _NPAIR - 1, lambda: start_in(even + 3, 1), lambda: None)
        return carry

    lax.fori_loop(0, _NPAIR, pair_body, 0)
    wait_out(0)
    wait_out(1)


def kernel(a, b, c):
    return _median_sc(a, b, c)


# speedup vs baseline: 9.8936x; 1.0006x over previous
import functools

import jax
import jax.numpy as jnp
from jax import lax
from jax.experimental import pallas as pl
from jax.experimental.pallas import tpu as pltpu
from jax.experimental.pallas import tpu_sc as plsc

_L = 16
_NC = 2
_NS = 16
_NW = _NC * _NS

_ROWS = 64
_COLS = 32768
_RB = 8
_NRB = _ROWS // _RB
_WPR = _NW // _NRB
_WCOLS = _COLS // _WPR
_CHUNK = 1024
_NCHUNK = _WCOLS // _CHUNK
_NPAIR = _NCHUNK // 2

_mesh = plsc.VectorSubcoreMesh(core_axis_name="c", subcore_axis_name="s")


@functools.partial(
    pl.kernel,
    mesh=_mesh,
    out_type=jax.ShapeDtypeStruct((_ROWS, _COLS), jnp.float32),
    scratch_types=(
        [pltpu.VMEM((_RB, _CHUNK), jnp.float32)] * 8
        + [pltpu.SemaphoreType.DMA] * 4
    ),
)
def _median_sc(a_hbm, b_hbm, c_hbm, out_hbm,
               av0, av1, bv0, bv1, cv0, cv1, ov0, ov1,
               si0, si1, so0, so1):
    wid = lax.axis_index("s") * _NC + lax.axis_index("c")
    r0 = (wid // _WPR) * _RB
    c0 = (wid % _WPR) * _WCOLS

    ins = [(av0, bv0, cv0, si0), (av1, bv1, cv1, si1)]
    outs = [(ov0, so0), (ov1, so1)]

    def hbm_slice(ref, ci):
        return ref.at[pl.ds(r0, _RB), pl.ds(c0 + ci * _CHUNK, _CHUNK)]

    def start_in(ci, p):
        av, bv, cv, si = ins[p]
        pltpu.async_copy(hbm_slice(a_hbm, ci), av, si)
        pltpu.async_copy(hbm_slice(b_hbm, ci), bv, si)
        pltpu.async_copy(hbm_slice(c_hbm, ci), cv, si)

    def wait_in(p):
        av, bv, cv, si = ins[p]
        pltpu.make_async_copy(hbm_slice(a_hbm, 0), av, si).wait()
        pltpu.make_async_copy(hbm_slice(b_hbm, 0), bv, si).wait()
        pltpu.make_async_copy(hbm_slice(c_hbm, 0), cv, si).wait()

    def wait_out(p):
        ov, so = outs[p]
        pltpu.make_async_copy(ov, hbm_slice(out_hbm, 0), so).wait()

    def compute(p, ci):
        av, bv, cv, _ = ins[p]
        ov, so = outs[p]
        @plsc.parallel_loop(0, _RB, 1)
        def row_body(r):
            @plsc.parallel_loop(0, _CHUNK, _L, unroll=8)
            def vec_body(s):
                x = av[r, pl.ds(s, _L)]
                y = bv[r, pl.ds(s, _L)]
                z = cv[r, pl.ds(s, _L)]
                lo = jnp.minimum(x, y)
                hi = jnp.maximum(x, y)
                ov[r, pl.ds(s, _L)] = jnp.maximum(lo, jnp.minimum(hi, z))
        pltpu.async_copy(ov, hbm_slice(out_hbm, ci), so)

    start_in(0, 0)
    start_in(1, 1)

    def pair_body(k, carry):
        even = 2 * k
        wait_in(0)
        lax.cond(k > 0, lambda: wait_out(0), lambda: None)
        compute(0, even)
        lax.cond(k < _NPAIR - 1, lambda: start_in(even + 2, 0), lambda: None)
        wait_in(1)
        lax.cond(k > 0, lambda: wait_out(1), lambda: None)
        compute(1, even + 1)
        lax.cond(k < _NPAIR - 1, lambda: start_in(even + 3, 1), lambda: None)
        return carry

    lax.fori_loop(0, _NPAIR, pair_body, 0)
    wait_out(0)
    wait_out(1)


def kernel(a, b, c):
    return _median_sc(a, b, c)
